# baseline (speedup 1.0000x reference)
import functools
import jax
import jax.numpy as jnp
from jax import lax
from jax.experimental import pallas as pl
from jax.experimental.pallas import tpu as pltpu
from jax.experimental.pallas import tpu_sc as plsc

N = 10000
E = 320000
D = 128
H = 128
C = 16
G = 64

NPAD = 10112
RPT = NPAD // 16
NTILES = 32
CHUNK = 128
CH_F = 104
CH_S = 56
TOTCH = 16 * (CH_F + CH_S)
DEG_CH = TOTCH // NTILES
EPAD = TOTCH * CHUNK

_MESH = plsc.VectorSubcoreMesh(core_axis_name="c", subcore_axis_name="s")


@functools.partial(
    pl.kernel,
    mesh=_MESH,
    out_type=jax.ShapeDtypeStruct((2, NPAD, D), jnp.float32),
    scratch_types=[
        pltpu.VMEM((CH_F, CHUNK), jnp.int32),
        pltpu.VMEM((CH_F, CHUNK), jnp.int32),
        pltpu.VMEM((CHUNK, D), jnp.float32),
        pltpu.VMEM_SHARED((NPAD, D), jnp.float32),
        pltpu.SemaphoreType.DMA,
    ],
)
def _sc_seg_sum(xa, srcp, dstp, zeros, out, sidx, didx, rows, acc, sem):
  c = lax.axis_index("c")
  s = lax.axis_index("s")
  pltpu.sync_copy(zeros, acc.at[pl.ds(s * RPT, RPT)])

  def run(base, n_ch):
    pltpu.sync_copy(srcp.at[pl.ds(base, n_ch)], sidx.at[pl.ds(0, n_ch)])
    pltpu.sync_copy(dstp.at[pl.ds(base, n_ch)], didx.at[pl.ds(0, n_ch)])
    plsc.subcore_barrier()

    def body(j, carry):
      pltpu.async_copy(xa.at[sidx.at[j]], rows, sem).wait()
      pltpu.sync_copy(rows, acc.at[didx.at[j]], add=True)
      return carry

    lax.fori_loop(0, n_ch, body, 0)

  @pl.when(c == 0)
  def _():
    run(s * CH_F, CH_F)

  @pl.when(c == 1)
  def _():
    run(16 * CH_F + s * CH_S, CH_S)

  plsc.subcore_barrier()
  pltpu.sync_copy(acc.at[pl.ds(s * RPT, RPT)], out.at[c, pl.ds(s * RPT, RPT)])


@functools.partial(
    pl.kernel,
    mesh=_MESH,
    out_type=jax.ShapeDtypeStruct((2, NPAD, D), jnp.float32),
    scratch_types=[
        pltpu.VMEM((DEG_CH, CHUNK), jnp.int32),
        pltpu.VMEM((CHUNK, D), jnp.float32),
        pltpu.VMEM_SHARED((NPAD, D), jnp.float32),
    ],
)
def _sc_degree(ones_hbm, dstp, zeros, out, didx, ones_v, acc):
  c = lax.axis_index("c")
  s = lax.axis_index("s")
  wid = s * 2 + c
  pltpu.sync_copy(zeros, acc.at[pl.ds(s * RPT, RPT)])
  pltpu.sync_copy(ones_hbm, ones_v)
  pltpu.sync_copy(dstp.at[pl.ds(wid * DEG_CH, DEG_CH)], didx)
  plsc.subcore_barrier()

  def body(j, carry):
    pltpu.sync_copy(ones_v, acc.at[didx.at[j]], add=True)
    return carry

  lax.fori_loop(0, DEG_CH, body, 0)
  plsc.subcore_barrier()
  pltpu.sync_copy(acc.at[pl.ds(s * RPT, RPT)], out.at[c, pl.ds(s * RPT, RPT)])


BLK = 400
NBLK = N // BLK


def _tc1_body(p0, p1, dg0, dg1, x, wl, wr, b, h_ref, dinv_ref):
  deg = dg0[0, :, :1] + dg1[0, :, :1]
  dinv = 1.0 / jnp.maximum(deg, 1.0)
  mean = (p0[0] + p1[0]) * dinv
  h = jnp.dot(mean, wl[...], preferred_element_type=jnp.float32)
  h = h + jnp.dot(x[...], wr[...], preferred_element_type=jnp.float32)
  h = h + b[...]
  h_ref[...] = jnp.maximum(h, 0.0)
  dinv_ref[...] = jnp.broadcast_to(dinv, (BLK, 8))


def _tc2_body(q0, q1, h1, dinv, bat, wl, wr, b, wc, bc,
              out_ref, pool_scr, cnt_scr):
  i = pl.program_id(0)

  @pl.when(i == 0)
  def _():
    pool_scr[...] = jnp.zeros_like(pool_scr)
    cnt_scr[...] = jnp.zeros_like(cnt_scr)

  mean = (q0[0] + q1[0]) * dinv[...][:, :1]
  h = jnp.dot(mean, wl[...], preferred_element_type=jnp.float32)
  h = h + jnp.dot(h1[...], wr[...], preferred_element_type=jnp.float32)
  h = h + b[...]
  bv = bat[...].reshape(1, BLK)
  onehot = (lax.broadcasted_iota(jnp.int32, (G, BLK), 0) == bv
            ).astype(jnp.float32)
  pool_scr[...] += jnp.dot(onehot, h, preferred_element_type=jnp.float32)
  cnt_scr[...] += jnp.broadcast_to(
      jnp.sum(onehot, axis=1, keepdims=True), (G, D))

  @pl.when(i == NBLK - 1)
  def _():
    gmean = pool_scr[...] / jnp.maximum(cnt_scr[...], 1.0)
    out_ref[...] = jnp.dot(gmean, wc[...],
                           preferred_element_type=jnp.float32) + bc[...]


def _part_spec(core):
  return pl.BlockSpec((1, BLK, D), lambda i, core=core: (core, i, 0))


def _tc1(p, dg, x, wl, wr, b):
  return pl.pallas_call(
      _tc1_body,
      grid=(NBLK,),
      in_specs=[
          _part_spec(0),
          _part_spec(1),
          _part_spec(0),
          _part_spec(1),
          pl.BlockSpec((BLK, D), lambda i: (i, 0)),
          pl.BlockSpec((D, H), lambda i: (0, 0)),
          pl.BlockSpec((D, H), lambda i: (0, 0)),
          pl.BlockSpec((1, H), lambda i: (0, 0)),
      ],
      out_specs=[
          pl.BlockSpec((BLK, H), lambda i: (i, 0)),
          pl.BlockSpec((BLK, 8), lambda i: (i, 0)),
      ],
      out_shape=[
          jax.ShapeDtypeStruct((N, H), jnp.float32),
          jax.ShapeDtypeStruct((N, 8), jnp.float32),
      ],
  )(p, p, dg, dg, x, wl, wr, b)


def _tc2(q, h1, dinv, bat3, wl, wr, b, wc, bc):
  return pl.pallas_call(
      _tc2_body,
      grid=(NBLK,),
      in_specs=[
          _part_spec(0),
          _part_spec(1),
          pl.BlockSpec((BLK, H), lambda i: (i, 0)),
          pl.BlockSpec((BLK, 8), lambda i: (i, 0)),
          pl.BlockSpec((1, 1, BLK), lambda i: (i, 0, 0)),
          pl.BlockSpec((H, H), lambda i: (0, 0)),
          pl.BlockSpec((H, H), lambda i: (0, 0)),
          pl.BlockSpec((1, H), lambda i: (0, 0)),
          pl.BlockSpec((H, C), lambda i: (0, 0)),
          pl.BlockSpec((1, C), lambda i: (0, 0)),
      ],
      out_specs=pl.BlockSpec((G, C), lambda i: (0, 0)),
      out_shape=jax.ShapeDtypeStruct((G, C), jnp.float32),
      scratch_shapes=[
          pltpu.VMEM((G, D), jnp.float32),
          pltpu.VMEM((G, D), jnp.float32),
      ],
  )(q, q, h1, dinv, bat3, wl, wr, b, wc, bc)


def kernel(x, edge_index, batch, Wl1, Wr1, b1, Wl2, Wr2, b2, Wc, bc):
  f32 = jnp.float32
  pad_src = jnp.zeros((EPAD - E,), jnp.int32)
  pad_dst = jnp.full((EPAD - E,), N, jnp.int32)
  srcp = jnp.concatenate([edge_index[0], pad_src]).reshape(TOTCH, CHUNK)
  dstp = jnp.concatenate([edge_index[1], pad_dst]).reshape(TOTCH, CHUNK)
  zeros_d = jnp.zeros((RPT, D), f32)
  ones_d = jnp.ones((CHUNK, D), f32)

  dg = _sc_degree(ones_d, dstp, zeros_d)
  p = _sc_seg_sum(x, srcp, dstp, zeros_d)
  h1, dinv = _tc1(p, dg, x, Wl1, Wr1, b1.reshape(1, H))

  q = _sc_seg_sum(h1, srcp, dstp, zeros_d)
  bat3 = batch.reshape(NBLK, 1, BLK)
  return _tc2(q, h1, dinv, bat3,
              Wl2, Wr2, b2.reshape(1, H), Wc, bc.reshape(1, C))


# speedup vs baseline: 1.0485x; 1.0485x over previous
import functools
import jax
import jax.numpy as jnp
from jax import lax
from jax.experimental import pallas as pl
from jax.experimental.pallas import tpu as pltpu
from jax.experimental.pallas import tpu_sc as plsc

N = 10000
E = 320000
D = 128
H = 128
C = 16
G = 64

NPAD = 10112
RPT = NPAD // 16
NTILES = 32
CHUNK = 128
CH_F = 104
CH_S = 56
TOTCH = 16 * (CH_F + CH_S)
DEG_CH = TOTCH // NTILES
EPAD = TOTCH * CHUNK

_MESH = plsc.VectorSubcoreMesh(core_axis_name="c", subcore_axis_name="s")


@functools.partial(
    pl.kernel,
    mesh=_MESH,
    out_type=jax.ShapeDtypeStruct((2, NPAD, D), jnp.float32),
    scratch_types=[
        pltpu.VMEM((CH_F, CHUNK), jnp.int32),
        pltpu.VMEM((CH_F, CHUNK), jnp.int32),
        pltpu.VMEM((CHUNK, D), jnp.float32),
        pltpu.VMEM_SHARED((NPAD, D), jnp.float32),
        pltpu.SemaphoreType.DMA,
    ],
)
def _sc_seg_sum(xa, srcp, dstp, zeros, out, sidx, didx, rows, acc, sem):
  c = lax.axis_index("c")
  s = lax.axis_index("s")
  pltpu.sync_copy(zeros, acc.at[pl.ds(s * RPT, RPT)])

  def run(base, n_ch):
    pltpu.sync_copy(srcp.at[pl.ds(base, n_ch)], sidx.at[pl.ds(0, n_ch)])
    pltpu.sync_copy(dstp.at[pl.ds(base, n_ch)], didx.at[pl.ds(0, n_ch)])
    plsc.subcore_barrier()

    def body(j, carry):
      pltpu.async_copy(xa.at[sidx.at[j]], rows, sem).wait()
      pltpu.sync_copy(rows, acc.at[didx.at[j]], add=True)
      return carry

    lax.fori_loop(0, n_ch, body, 0)

  @pl.when(c == 1)
  def _():
    run(s * CH_F, CH_F)

  @pl.when(c == 0)
  def _():
    run(16 * CH_F + s * CH_S, CH_S)

  plsc.subcore_barrier()
  pltpu.sync_copy(acc.at[pl.ds(s * RPT, RPT)], out.at[c, pl.ds(s * RPT, RPT)])


@functools.partial(
    pl.kernel,
    mesh=_MESH,
    out_type=jax.ShapeDtypeStruct((2, NPAD, D), jnp.float32),
    scratch_types=[
        pltpu.VMEM((DEG_CH, CHUNK), jnp.int32),
        pltpu.VMEM((CHUNK, D), jnp.float32),
        pltpu.VMEM_SHARED((NPAD, D), jnp.float32),
    ],
)
def _sc_degree(ones_hbm, dstp, zeros, out, didx, ones_v, acc):
  c = lax.axis_index("c")
  s = lax.axis_index("s")
  wid = s * 2 + c
  pltpu.sync_copy(zeros, acc.at[pl.ds(s * RPT, RPT)])
  pltpu.sync_copy(ones_hbm, ones_v)
  pltpu.sync_copy(dstp.at[pl.ds(wid * DEG_CH, DEG_CH)], didx)
  plsc.subcore_barrier()

  def body(j, carry):
    pltpu.sync_copy(ones_v, acc.at[didx.at[j]], add=True)
    return carry

  lax.fori_loop(0, DEG_CH, body, 0)
  plsc.subcore_barrier()
  pltpu.sync_copy(acc.at[pl.ds(s * RPT, RPT)], out.at[c, pl.ds(s * RPT, RPT)])


BLK = 400
NBLK = N // BLK


def _tc1_body(p0, p1, dg0, dg1, x, wl, wr, b, h_ref, dinv_ref):
  deg = dg0[0, :, :1] + dg1[0, :, :1]
  dinv = 1.0 / jnp.maximum(deg, 1.0)
  mean = (p0[0] + p1[0]) * dinv
  h = jnp.dot(mean, wl[...], preferred_element_type=jnp.float32)
  h = h + jnp.dot(x[...], wr[...], preferred_element_type=jnp.float32)
  h = h + b[...]
  h_ref[...] = jnp.maximum(h, 0.0)
  dinv_ref[...] = jnp.broadcast_to(dinv, (BLK, 8))


def _tc2_body(q0, q1, h1, dinv, bat, wl, wr, b, wc, bc,
              out_ref, pool_scr, cnt_scr):
  i = pl.program_id(0)

  @pl.when(i == 0)
  def _():
    pool_scr[...] = jnp.zeros_like(pool_scr)
    cnt_scr[...] = jnp.zeros_like(cnt_scr)

  mean = (q0[0] + q1[0]) * dinv[...][:, :1]
  h = jnp.dot(mean, wl[...], preferred_element_type=jnp.float32)
  h = h + jnp.dot(h1[...], wr[...], preferred_element_type=jnp.float32)
  h = h + b[...]
  bv = bat[...].reshape(1, BLK)
  onehot = (lax.broadcasted_iota(jnp.int32, (G, BLK), 0) == bv
            ).astype(jnp.float32)
  pool_scr[...] += jnp.dot(onehot, h, preferred_element_type=jnp.float32)
  cnt_scr[...] += jnp.broadcast_to(
      jnp.sum(onehot, axis=1, keepdims=True), (G, D))

  @pl.when(i == NBLK - 1)
  def _():
    gmean = pool_scr[...] / jnp.maximum(cnt_scr[...], 1.0)
    out_ref[...] = jnp.dot(gmean, wc[...],
                           preferred_element_type=jnp.float32) + bc[...]


def _part_spec(core):
  return pl.BlockSpec((1, BLK, D), lambda i, core=core: (core, i, 0))


def _tc1(p, dg, x, wl, wr, b):
  return pl.pallas_call(
      _tc1_body,
      grid=(NBLK,),
      in_specs=[
          _part_spec(0),
          _part_spec(1),
          _part_spec(0),
          _part_spec(1),
          pl.BlockSpec((BLK, D), lambda i: (i, 0)),
          pl.BlockSpec((D, H), lambda i: (0, 0)),
          pl.BlockSpec((D, H), lambda i: (0, 0)),
          pl.BlockSpec((1, H), lambda i: (0, 0)),
      ],
      out_specs=[
          pl.BlockSpec((BLK, H), lambda i: (i, 0)),
          pl.BlockSpec((BLK, 8), lambda i: (i, 0)),
      ],
      out_shape=[
          jax.ShapeDtypeStruct((N, H), jnp.float32),
          jax.ShapeDtypeStruct((N, 8), jnp.float32),
      ],
  )(p, p, dg, dg, x, wl, wr, b)


def _tc2(q, h1, dinv, bat3, wl, wr, b, wc, bc):
  return pl.pallas_call(
      _tc2_body,
      grid=(NBLK,),
      in_specs=[
          _part_spec(0),
          _part_spec(1),
          pl.BlockSpec((BLK, H), lambda i: (i, 0)),
          pl.BlockSpec((BLK, 8), lambda i: (i, 0)),
          pl.BlockSpec((1, 1, BLK), lambda i: (i, 0, 0)),
          pl.BlockSpec((H, H), lambda i: (0, 0)),
          pl.BlockSpec((H, H), lambda i: (0, 0)),
          pl.BlockSpec((1, H), lambda i: (0, 0)),
          pl.BlockSpec((H, C), lambda i: (0, 0)),
          pl.BlockSpec((1, C), lambda i: (0, 0)),
      ],
      out_specs=pl.BlockSpec((G, C), lambda i: (0, 0)),
      out_shape=jax.ShapeDtypeStruct((G, C), jnp.float32),
      scratch_shapes=[
          pltpu.VMEM((G, D), jnp.float32),
          pltpu.VMEM((G, D), jnp.float32),
      ],
  )(q, q, h1, dinv, bat3, wl, wr, b, wc, bc)


def kernel(x, edge_index, batch, Wl1, Wr1, b1, Wl2, Wr2, b2, Wc, bc):
  f32 = jnp.float32
  pad_src = jnp.zeros((EPAD - E,), jnp.int32)
  pad_dst = jnp.full((EPAD - E,), N, jnp.int32)
  srcp = jnp.concatenate([edge_index[0], pad_src]).reshape(TOTCH, CHUNK)
  dstp = jnp.concatenate([edge_index[1], pad_dst]).reshape(TOTCH, CHUNK)
  zeros_d = jnp.zeros((RPT, D), f32)
  ones_d = jnp.ones((CHUNK, D), f32)

  dg = _sc_degree(ones_d, dstp, zeros_d)
  p = _sc_seg_sum(x, srcp, dstp, zeros_d)
  h1, dinv = _tc1(p, dg, x, Wl1, Wr1, b1.reshape(1, H))

  q = _sc_seg_sum(h1, srcp, dstp, zeros_d)
  bat3 = batch.reshape(NBLK, 1, BLK)
  return _tc2(q, h1, dinv, bat3,
              Wl2, Wr2, b2.reshape(1, H), Wc, bc.reshape(1, C))


# speedup vs baseline: 1.3033x; 1.2430x over previous
import functools
import jax
import jax.numpy as jnp
from jax import lax
from jax.experimental import pallas as pl
from jax.experimental.pallas import tpu as pltpu
from jax.experimental.pallas import tpu_sc as plsc

N = 10000
E = 320000
D = 128
H = 128
C = 16
G = 64

NPAD = 10112
RPT = NPAD // 16
NTILES = 32
CHUNK = 128
EPT_CH = 79
EPT = EPT_CH * CHUNK
EPAD = EPT * NTILES

_MESH = plsc.VectorSubcoreMesh(core_axis_name="c", subcore_axis_name="s")


@functools.partial(
    pl.kernel,
    mesh=_MESH,
    out_type=jax.ShapeDtypeStruct((2, NPAD, D), jnp.float32),
    scratch_types=[
        pltpu.VMEM((EPT_CH, CHUNK), jnp.int32),
        pltpu.VMEM((EPT_CH, CHUNK), jnp.int32),
        pltpu.VMEM((CHUNK, D), jnp.float32),
        pltpu.VMEM_SHARED((NPAD, D), jnp.float32),
        pltpu.SemaphoreType.DMA,
    ],
)
def _sc_seg_sum(xa, srcp, dstp, zeros, out, sidx, didx, rows, acc, sem):
  c = lax.axis_index("c")
  s = lax.axis_index("s")
  wid = s * 2 + c
  pltpu.sync_copy(zeros, acc.at[pl.ds(s * RPT, RPT)])
  pltpu.sync_copy(srcp.at[wid], sidx)
  pltpu.sync_copy(dstp.at[wid], didx)
  plsc.subcore_barrier()

  def body(j, carry):
    pltpu.async_copy(xa.at[sidx.at[j]], rows, sem).wait()
    pltpu.sync_copy(rows, acc.at[didx.at[j]], add=True)
    return carry

  lax.fori_loop(0, EPT_CH, body, 0)
  plsc.subcore_barrier()
  pltpu.sync_copy(acc.at[pl.ds(s * RPT, RPT)], out.at[c, pl.ds(s * RPT, RPT)])


@functools.partial(
    pl.kernel,
    mesh=_MESH,
    out_type=jax.ShapeDtypeStruct((2, NPAD, D), jnp.float32),
    scratch_types=[
        pltpu.VMEM((EPT_CH, CHUNK), jnp.int32),
        pltpu.VMEM((CHUNK, D), jnp.float32),
        pltpu.VMEM_SHARED((NPAD, D), jnp.float32),
    ],
)
def _sc_degree(ones_hbm, dstp, zeros, out, didx, ones_v, acc):
  c = lax.axis_index("c")
  s = lax.axis_index("s")
  wid = s * 2 + c
  pltpu.sync_copy(zeros, acc.at[pl.ds(s * RPT, RPT)])
  pltpu.sync_copy(ones_hbm, ones_v)
  pltpu.sync_copy(dstp.at[wid], didx)
  plsc.subcore_barrier()

  def body(j, carry):
    pltpu.sync_copy(ones_v, acc.at[didx.at[j]], add=True)
    return carry

  lax.fori_loop(0, EPT_CH, body, 0)
  plsc.subcore_barrier()
  pltpu.sync_copy(acc.at[pl.ds(s * RPT, RPT)], out.at[c, pl.ds(s * RPT, RPT)])


BLK = 400
NBLK = N // BLK


def _tc1_body(p0, p1, dg0, dg1, x, wl, wr, b, h_ref, dinv_ref):
  deg = dg0[0, :, :1] + dg1[0, :, :1]
  dinv = 1.0 / jnp.maximum(deg, 1.0)
  mean = (p0[0] + p1[0]) * dinv
  h = jnp.dot(mean, wl[...], preferred_element_type=jnp.float32)
  h = h + jnp.dot(x[...], wr[...], preferred_element_type=jnp.float32)
  h = h + b[...]
  h_ref[...] = jnp.maximum(h, 0.0)
  dinv_ref[...] = jnp.broadcast_to(dinv, (BLK, 8))


def _tc2_body(q0, q1, h1, dinv, bat, wl, wr, b, wc, bc,
              out_ref, pool_scr, cnt_scr):
  i = pl.program_id(0)

  @pl.when(i == 0)
  def _():
    pool_scr[...] = jnp.zeros_like(pool_scr)
    cnt_scr[...] = jnp.zeros_like(cnt_scr)

  mean = (q0[0] + q1[0]) * dinv[...][:, :1]
  h = jnp.dot(mean, wl[...], preferred_element_type=jnp.float32)
  h = h + jnp.dot(h1[...], wr[...], preferred_element_type=jnp.float32)
  h = h + b[...]
  bv = bat[...].reshape(1, BLK)
  onehot = (lax.broadcasted_iota(jnp.int32, (G, BLK), 0) == bv
            ).astype(jnp.float32)
  pool_scr[...] += jnp.dot(onehot, h, preferred_element_type=jnp.float32)
  cnt_scr[...] += jnp.broadcast_to(
      jnp.sum(onehot, axis=1, keepdims=True), (G, D))

  @pl.when(i == NBLK - 1)
  def _():
    gmean = pool_scr[...] / jnp.maximum(cnt_scr[...], 1.0)
    out_ref[...] = jnp.dot(gmean, wc[...],
                           preferred_element_type=jnp.float32) + bc[...]


def _part_spec(core):
  return pl.BlockSpec((1, BLK, D), lambda i, core=core: (core, i, 0))


def _tc1(p, dg, x, wl, wr, b):
  return pl.pallas_call(
      _tc1_body,
      grid=(NBLK,),
      in_specs=[
          _part_spec(0),
          _part_spec(1),
          _part_spec(0),
          _part_spec(1),
          pl.BlockSpec((BLK, D), lambda i: (i, 0)),
          pl.BlockSpec((D, H), lambda i: (0, 0)),
          pl.BlockSpec((D, H), lambda i: (0, 0)),
          pl.BlockSpec((1, H), lambda i: (0, 0)),
      ],
      out_specs=[
          pl.BlockSpec((BLK, H), lambda i: (i, 0)),
          pl.BlockSpec((BLK, 8), lambda i: (i, 0)),
      ],
      out_shape=[
          jax.ShapeDtypeStruct((N, H), jnp.float32),
          jax.ShapeDtypeStruct((N, 8), jnp.float32),
      ],
  )(p, p, dg, dg, x, wl, wr, b)


def _tc2(q, h1, dinv, bat3, wl, wr, b, wc, bc):
  return pl.pallas_call(
      _tc2_body,
      grid=(NBLK,),
      in_specs=[
          _part_spec(0),
          _part_spec(1),
          pl.BlockSpec((BLK, H), lambda i: (i, 0)),
          pl.BlockSpec((BLK, 8), lambda i: (i, 0)),
          pl.BlockSpec((1, 1, BLK), lambda i: (i, 0, 0)),
          pl.BlockSpec((H, H), lambda i: (0, 0)),
          pl.BlockSpec((H, H), lambda i: (0, 0)),
          pl.BlockSpec((1, H), lambda i: (0, 0)),
          pl.BlockSpec((H, C), lambda i: (0, 0)),
          pl.BlockSpec((1, C), lambda i: (0, 0)),
      ],
      out_specs=pl.BlockSpec((G, C), lambda i: (0, 0)),
      out_shape=jax.ShapeDtypeStruct((G, C), jnp.float32),
      scratch_shapes=[
          pltpu.VMEM((G, D), jnp.float32),
          pltpu.VMEM((G, D), jnp.float32),
      ],
  )(q, q, h1, dinv, bat3, wl, wr, b, wc, bc)


def kernel(x, edge_index, batch, Wl1, Wr1, b1, Wl2, Wr2, b2, Wc, bc):
  f32 = jnp.float32
  pad_src = jnp.zeros((EPAD - E,), jnp.int32)
  pad_dst = jnp.full((EPAD - E,), N, jnp.int32)
  srcp = jnp.concatenate([edge_index[0], pad_src]).reshape(
      NTILES, EPT_CH, CHUNK)
  dstp = jnp.concatenate([edge_index[1], pad_dst]).reshape(
      NTILES, EPT_CH, CHUNK)
  zeros_d = jnp.zeros((RPT, D), f32)
  ones_d = jnp.ones((CHUNK, D), f32)

  dg = _sc_degree(ones_d, dstp, zeros_d)
  p = _sc_seg_sum(x, srcp, dstp, zeros_d)
  h1, dinv = _tc1(p, dg, x, Wl1, Wr1, b1.reshape(1, H))

  q = _sc_seg_sum(h1, srcp, dstp, zeros_d)
  bat3 = batch.reshape(NBLK, 1, BLK)
  return _tc2(q, h1, dinv, bat3,
              Wl2, Wr2, b2.reshape(1, H), Wc, bc.reshape(1, C))
